# Initial kernel scaffold; baseline (speedup 1.0000x reference)
import jax
import jax.numpy as jnp
from jax.experimental import pallas as pl


def kernel(h, edge_index, e, snorm_n, snorm_e, W_emb, b_emb, W1, b1, W2, b2, Wp, bp):
    raise NotImplementedError("write your pallas kernel here")


# speedup vs baseline: 4.9342x; 4.9342x over previous
import functools

import jax
import jax.numpy as jnp
from jax import lax
from jax.experimental import pallas as pl
from jax.experimental.pallas import tpu as pltpu
from jax.experimental.pallas import tpu_sc as plsc

N = 10000
E = 160000
HID = 256
NCLS = 40
DP = 48
NPAD = 10240
NC = 2
NS = 16
NW = NC * NS
CHUNK = 128
EP = 163840
EPT = EP // NW
GCH = EPT // CHUNK
NROWS_T = NPAD // NS

_mesh = plsc.VectorSubcoreMesh(core_axis_name="c", subcore_axis_name="s")
_sc_params = pltpu.CompilerParams(use_tc_tiling_on_sc=False)


def _deg_body(srcp_hbm, dstp_hbm, out_hbm, idx_v, ones_v, zb_v, acc_o, acc_i):
    cid = lax.axis_index("c")
    sid = lax.axis_index("s")
    wid = sid * NC + cid
    for k in range(CHUNK // 16):
        ones_v[pl.ds(k * 16, 16)] = jnp.full((16,), 1.0, jnp.float32)
    for k in range(NROWS_T // 16):
        zb_v[pl.ds(k * 16, 16)] = jnp.zeros((16,), jnp.float32)
    base_n = sid * NROWS_T
    pltpu.sync_copy(zb_v, acc_o.at[pl.ds(base_n, NROWS_T)])
    pltpu.sync_copy(zb_v, acc_i.at[pl.ds(base_n, NROWS_T)])
    plsc.subcore_barrier()

    def step(g, carry):
        base = wid * EPT + g * CHUNK
        pltpu.sync_copy(srcp_hbm.at[pl.ds(base, CHUNK)], idx_v)
        pltpu.sync_copy(ones_v, acc_o.at[idx_v], add=True)
        pltpu.sync_copy(dstp_hbm.at[pl.ds(base, CHUNK)], idx_v)
        pltpu.sync_copy(ones_v, acc_i.at[idx_v], add=True)
        return carry

    lax.fori_loop(0, GCH, step, 0)
    plsc.subcore_barrier()
    pltpu.sync_copy(acc_o.at[pl.ds(base_n, NROWS_T)], zb_v)
    pltpu.sync_copy(zb_v, out_hbm.at[cid, 0, pl.ds(base_n, NROWS_T)])
    pltpu.sync_copy(acc_i.at[pl.ds(base_n, NROWS_T)], zb_v)
    pltpu.sync_copy(zb_v, out_hbm.at[cid, 1, pl.ds(base_n, NROWS_T)])


_deg_call = pl.kernel(
    _deg_body,
    out_type=jax.ShapeDtypeStruct((NC, 2, NPAD), jnp.float32),
    mesh=_mesh,
    scratch_types=[
        pltpu.VMEM((CHUNK,), jnp.int32),
        pltpu.VMEM((CHUNK,), jnp.float32),
        pltpu.VMEM((NROWS_T,), jnp.float32),
        pltpu.VMEM_SHARED((NPAD,), jnp.float32),
        pltpu.VMEM_SHARED((NPAD,), jnp.float32),
    ],
    compiler_params=_sc_params,
)


def _hop_body(zs_hbm, srcp_hbm, dstp_hbm, out_hbm,
              sidx_v, didx_v, rows_v, acc, gsem):
    cid = lax.axis_index("c")
    sid = lax.axis_index("s")
    wid = sid * NC + cid

    def zrow(r, carry):
        for k in range(DP // 16):
            rows_v[r, pl.ds(k * 16, 16)] = jnp.zeros((16,), jnp.float32)
        return carry

    lax.fori_loop(0, CHUNK, zrow, 0)
    base_n = sid * NROWS_T
    for k in range(NROWS_T // CHUNK):
        pltpu.sync_copy(rows_v, acc.at[pl.ds(base_n + k * CHUNK, CHUNK), :])
    plsc.subcore_barrier()

    def step(g, carry):
        base = wid * EPT + g * CHUNK
        pltpu.sync_copy(srcp_hbm.at[pl.ds(base, CHUNK)], sidx_v)
        pltpu.sync_copy(dstp_hbm.at[pl.ds(base, CHUNK)], didx_v)
        pltpu.async_copy(zs_hbm.at[sidx_v], rows_v, gsem).wait()
        pltpu.sync_copy(rows_v, acc.at[didx_v], add=True)
        return carry

    lax.fori_loop(0, GCH, step, 0)
    plsc.subcore_barrier()
    for k in range(NROWS_T // CHUNK):
        pltpu.sync_copy(acc.at[pl.ds(base_n + k * CHUNK, CHUNK), :], rows_v)
        pltpu.sync_copy(rows_v,
                        out_hbm.at[cid, pl.ds(base_n + k * CHUNK, CHUNK), :])


_hop_call = pl.kernel(
    _hop_body,
    out_type=jax.ShapeDtypeStruct((NC, NPAD, DP), jnp.float32),
    mesh=_mesh,
    scratch_types=[
        pltpu.VMEM((CHUNK,), jnp.int32),
        pltpu.VMEM((CHUNK,), jnp.int32),
        pltpu.VMEM((CHUNK, DP), jnp.float32),
        pltpu.VMEM_SHARED((NPAD, DP), jnp.float32),
        pltpu.SemaphoreType.DMA,
    ],
    compiler_params=_sc_params,
)


BLK = 1000


def _mlp_body(h_ref, we_ref, be_ref, w1_ref, b1_ref, w2_ref, b2_ref, wp_ref,
              out_ref):
    x = jnp.dot(h_ref[...], we_ref[...], preferred_element_type=jnp.float32)
    x = x + be_ref[...]
    x = jnp.dot(x, w1_ref[...], preferred_element_type=jnp.float32) + b1_ref[...]
    x = jnp.maximum(x, 0.0)
    x = jnp.dot(x, w2_ref[...], preferred_element_type=jnp.float32) + b2_ref[...]
    out_ref[...] = jnp.dot(x, wp_ref[...], preferred_element_type=jnp.float32)


def _norms(degp):
    no = lax.rsqrt(jnp.maximum(degp[:, 0] + degp[:, 2], 1.0))
    ni = lax.rsqrt(jnp.maximum(degp[:, 1] + degp[:, 3], 1.0))
    return no, ni


def _scale_body(degp_ref, z0_ref, out_ref):
    no, _ = _norms(degp_ref[...])
    out_ref[...] = z0_ref[...] * no[:, None]


def _mid_body(p_ref, degp_ref, out_ref):
    no, ni = _norms(degp_ref[...])
    p = p_ref[...]
    out_ref[...] = (p[0] + p[1]) * (ni * no)[:, None]


def _fin_body(p_ref, degp_ref, bp_ref, out_ref):
    _, ni = _norms(degp_ref[...])
    p = p_ref[...]
    y = (p[0] + p[1]) * ni[:, None]
    out_ref[...] = y[:, :NCLS] + bp_ref[...]


_full = lambda *shape: pl.BlockSpec(shape, lambda i: (0,) * len(shape))
_degp_spec = pl.BlockSpec((BLK, 4), lambda i: (i, 0))
_part_spec = pl.BlockSpec((NC, BLK, DP), lambda i: (0, i, 0))

_mlp_call = pl.pallas_call(
    _mlp_body,
    grid=(N // BLK,),
    in_specs=[
        pl.BlockSpec((BLK, HID), lambda i: (i, 0)),
        _full(HID, HID), _full(1, HID),
        _full(HID, HID), _full(1, HID),
        _full(HID, HID), _full(1, HID),
        _full(HID, DP),
    ],
    out_specs=pl.BlockSpec((BLK, DP), lambda i: (i, 0)),
    out_shape=jax.ShapeDtypeStruct((N, DP), jnp.float32),
)

_scale_call = pl.pallas_call(
    _scale_body,
    grid=(N // BLK,),
    in_specs=[_degp_spec, pl.BlockSpec((BLK, DP), lambda i: (i, 0))],
    out_specs=pl.BlockSpec((BLK, DP), lambda i: (i, 0)),
    out_shape=jax.ShapeDtypeStruct((N, DP), jnp.float32),
)

_mid_call = pl.pallas_call(
    _mid_body,
    grid=(N // BLK,),
    in_specs=[_part_spec, _degp_spec],
    out_specs=pl.BlockSpec((BLK, DP), lambda i: (i, 0)),
    out_shape=jax.ShapeDtypeStruct((N, DP), jnp.float32),
)

_fin_call = pl.pallas_call(
    _fin_body,
    grid=(N // BLK,),
    in_specs=[_part_spec, _degp_spec, _full(1, NCLS)],
    out_specs=pl.BlockSpec((BLK, NCLS), lambda i: (i, 0)),
    out_shape=jax.ShapeDtypeStruct((N, NCLS), jnp.float32),
)


def kernel(h, edge_index, e, snorm_n, snorm_e,
           W_emb, b_emb, W1, b1, W2, b2, Wp, bp):
    del e, snorm_n, snorm_e
    src = edge_index[0]
    dst = edge_index[1]
    pad = EP - E
    pad_dummy = jnp.full((pad,), N, jnp.int32)
    srcp_deg = jnp.concatenate([src, pad_dummy])
    srcp_hop = jnp.concatenate([src, jnp.zeros((pad,), jnp.int32)])
    dstp = jnp.concatenate([dst, pad_dummy])
    Wp_pad = jnp.pad(Wp, ((0, 0), (0, DP - NCLS)))

    degp = _deg_call(srcp_deg, dstp)
    degp_t = degp.reshape(2 * NC, NPAD).T
    z0 = _mlp_call(h, W_emb, b_emb.reshape(1, HID), W1, b1.reshape(1, HID),
                   W2, b2.reshape(1, HID), Wp_pad)
    zs = _scale_call(degp_t, z0)
    p1 = _hop_call(zs, srcp_hop, dstp)
    zs2 = _mid_call(p1, degp_t)
    p2 = _hop_call(zs2, srcp_hop, dstp)
    return _fin_call(p2, degp_t, bp.reshape(1, NCLS))


# speedup vs baseline: 6.8980x; 1.3980x over previous
import functools

import jax
import jax.numpy as jnp
from jax import lax
from jax.experimental import pallas as pl
from jax.experimental.pallas import tpu as pltpu
from jax.experimental.pallas import tpu_sc as plsc

N = 10000
E = 160000
HID = 256
NCLS = 40
DP = 48
NPAD = 10240
NC = 2
NS = 16
NW = NC * NS
CHUNK = 128
EP = 163840
EPT = EP // NW
GCH = EPT // CHUNK
NROWS_T = NPAD // NS

_mesh = plsc.VectorSubcoreMesh(core_axis_name="c", subcore_axis_name="s")
_sc_params = pltpu.CompilerParams(use_tc_tiling_on_sc=False)


def _deg_body(srcp_hbm, dstp_hbm, out_hbm,
              sidx_v, didx_v, ones_v, zb_v, acc_o, acc_i, sem):
    cid = lax.axis_index("c")
    sid = lax.axis_index("s")
    wid = sid * NC + cid
    for k in range(CHUNK // 16):
        ones_v[pl.ds(k * 16, 16)] = jnp.full((16,), 1.0, jnp.float32)
    for k in range(NROWS_T // 16):
        zb_v[pl.ds(k * 16, 16)] = jnp.zeros((16,), jnp.float32)
    base_n = sid * NROWS_T
    pre = [pltpu.async_copy(srcp_hbm.at[pl.ds(wid * GCH, GCH), :], sidx_v, sem),
           pltpu.async_copy(dstp_hbm.at[pl.ds(wid * GCH, GCH), :], didx_v, sem)]
    pltpu.sync_copy(zb_v, acc_o.at[pl.ds(base_n, NROWS_T)])
    pltpu.sync_copy(zb_v, acc_i.at[pl.ds(base_n, NROWS_T)])
    for d in pre:
        d.wait()
    plsc.subcore_barrier()
    ds = []
    for g in range(GCH):
        ds.append(pltpu.async_copy(ones_v, acc_o.at[sidx_v.at[g]], sem,
                                   add=True))
        ds.append(pltpu.async_copy(ones_v, acc_i.at[didx_v.at[g]], sem,
                                   add=True))
    for d in ds:
        d.wait()
    plsc.subcore_barrier()
    pltpu.sync_copy(acc_o.at[pl.ds(base_n, NROWS_T)], zb_v)
    pltpu.sync_copy(zb_v, out_hbm.at[cid, 0, pl.ds(base_n, NROWS_T)])
    pltpu.sync_copy(acc_i.at[pl.ds(base_n, NROWS_T)], zb_v)
    pltpu.sync_copy(zb_v, out_hbm.at[cid, 1, pl.ds(base_n, NROWS_T)])


_deg_call = pl.kernel(
    _deg_body,
    out_type=jax.ShapeDtypeStruct((NC, 2, NPAD), jnp.float32),
    mesh=_mesh,
    scratch_types=[
        pltpu.VMEM((GCH, CHUNK), jnp.int32),
        pltpu.VMEM((GCH, CHUNK), jnp.int32),
        pltpu.VMEM((CHUNK,), jnp.float32),
        pltpu.VMEM((NROWS_T,), jnp.float32),
        pltpu.VMEM_SHARED((NPAD,), jnp.float32),
        pltpu.VMEM_SHARED((NPAD,), jnp.float32),
        pltpu.SemaphoreType.DMA,
    ],
    compiler_params=_sc_params,
)


NB = 8


def _hop_body(zs_hbm, srcp_hbm, dstp_hbm, out_hbm,
              sidx_v, didx_v, rows_v, acc, gsem, ssem):
    cid = lax.axis_index("c")
    sid = lax.axis_index("s")
    wid = sid * NC + cid

    pre = [pltpu.async_copy(srcp_hbm.at[pl.ds(wid * GCH, GCH), :], sidx_v,
                            gsem),
           pltpu.async_copy(dstp_hbm.at[pl.ds(wid * GCH, GCH), :], didx_v,
                            gsem)]

    def zrow(r, carry):
        for k in range(DP // 16):
            rows_v[0, r, pl.ds(k * 16, 16)] = jnp.zeros((16,), jnp.float32)
        return carry

    lax.fori_loop(0, CHUNK, zrow, 0)
    base_n = sid * NROWS_T
    for k in range(NROWS_T // CHUNK):
        pltpu.sync_copy(rows_v.at[0],
                        acc.at[pl.ds(base_n + k * CHUNK, CHUNK), :])
    for d in pre:
        d.wait()
    plsc.subcore_barrier()

    for grp in range(GCH // NB):
        gd = [pltpu.async_copy(zs_hbm.at[sidx_v.at[grp * NB + b]],
                               rows_v.at[b], gsem)
              for b in range(NB)]
        for d in gd:
            d.wait()
        sd = [pltpu.async_copy(rows_v.at[b],
                               acc.at[didx_v.at[grp * NB + b]], ssem,
                               add=True)
              for b in range(NB)]
        for d in sd:
            d.wait()

    plsc.subcore_barrier()
    wd = [pltpu.async_copy(acc.at[pl.ds(base_n + k * CHUNK, CHUNK), :],
                           rows_v.at[k], gsem)
          for k in range(NROWS_T // CHUNK)]
    for d in wd:
        d.wait()
    od = [pltpu.async_copy(rows_v.at[k],
                           out_hbm.at[cid, pl.ds(base_n + k * CHUNK, CHUNK),
                                      :], ssem)
          for k in range(NROWS_T // CHUNK)]
    for d in od:
        d.wait()


_hop_call = pl.kernel(
    _hop_body,
    out_type=jax.ShapeDtypeStruct((NC, NPAD, DP), jnp.float32),
    mesh=_mesh,
    scratch_types=[
        pltpu.VMEM((GCH, CHUNK), jnp.int32),
        pltpu.VMEM((GCH, CHUNK), jnp.int32),
        pltpu.VMEM((NB, CHUNK, DP), jnp.float32),
        pltpu.VMEM_SHARED((NPAD, DP), jnp.float32),
        pltpu.SemaphoreType.DMA,
        pltpu.SemaphoreType.DMA,
    ],
    compiler_params=_sc_params,
)


BLK = 1000


def _mlp_body(h_ref, we_ref, be_ref, w1_ref, b1_ref, w2_ref, b2_ref, wp_ref,
              out_ref):
    x = jnp.dot(h_ref[...], we_ref[...], preferred_element_type=jnp.float32)
    x = x + be_ref[...]
    x = jnp.dot(x, w1_ref[...], preferred_element_type=jnp.float32) + b1_ref[...]
    x = jnp.maximum(x, 0.0)
    x = jnp.dot(x, w2_ref[...], preferred_element_type=jnp.float32) + b2_ref[...]
    out_ref[...] = jnp.dot(x, wp_ref[...], preferred_element_type=jnp.float32)


def _norms(degp):
    no = lax.rsqrt(jnp.maximum(degp[:, 0] + degp[:, 2], 1.0))
    ni = lax.rsqrt(jnp.maximum(degp[:, 1] + degp[:, 3], 1.0))
    return no, ni


def _scale_body(degp_ref, z0_ref, out_ref):
    no, _ = _norms(degp_ref[...])
    out_ref[...] = z0_ref[...] * no[:, None]


def _mid_body(p_ref, degp_ref, out_ref):
    no, ni = _norms(degp_ref[...])
    p = p_ref[...]
    out_ref[...] = (p[0] + p[1]) * (ni * no)[:, None]


def _fin_body(p_ref, degp_ref, bp_ref, out_ref):
    _, ni = _norms(degp_ref[...])
    p = p_ref[...]
    y = (p[0] + p[1]) * ni[:, None]
    out_ref[...] = y[:, :NCLS] + bp_ref[...]


_full = lambda *shape: pl.BlockSpec(shape, lambda i: (0,) * len(shape))
_degp_spec = pl.BlockSpec((BLK, 4), lambda i: (i, 0))
_part_spec = pl.BlockSpec((NC, BLK, DP), lambda i: (0, i, 0))

_mlp_call = pl.pallas_call(
    _mlp_body,
    grid=(N // BLK,),
    in_specs=[
        pl.BlockSpec((BLK, HID), lambda i: (i, 0)),
        _full(HID, HID), _full(1, HID),
        _full(HID, HID), _full(1, HID),
        _full(HID, HID), _full(1, HID),
        _full(HID, DP),
    ],
    out_specs=pl.BlockSpec((BLK, DP), lambda i: (i, 0)),
    out_shape=jax.ShapeDtypeStruct((N, DP), jnp.float32),
)

_scale_call = pl.pallas_call(
    _scale_body,
    grid=(N // BLK,),
    in_specs=[_degp_spec, pl.BlockSpec((BLK, DP), lambda i: (i, 0))],
    out_specs=pl.BlockSpec((BLK, DP), lambda i: (i, 0)),
    out_shape=jax.ShapeDtypeStruct((N, DP), jnp.float32),
)

_mid_call = pl.pallas_call(
    _mid_body,
    grid=(N // BLK,),
    in_specs=[_part_spec, _degp_spec],
    out_specs=pl.BlockSpec((BLK, DP), lambda i: (i, 0)),
    out_shape=jax.ShapeDtypeStruct((N, DP), jnp.float32),
)

_fin_call = pl.pallas_call(
    _fin_body,
    grid=(N // BLK,),
    in_specs=[_part_spec, _degp_spec, _full(1, NCLS)],
    out_specs=pl.BlockSpec((BLK, NCLS), lambda i: (i, 0)),
    out_shape=jax.ShapeDtypeStruct((N, NCLS), jnp.float32),
)


def kernel(h, edge_index, e, snorm_n, snorm_e,
           W_emb, b_emb, W1, b1, W2, b2, Wp, bp):
    del e, snorm_n, snorm_e
    src = edge_index[0]
    dst = edge_index[1]
    pad = EP - E
    pad_dummy = jnp.full((pad,), N, jnp.int32)
    rows2d = (EP // CHUNK, CHUNK)
    srcp_deg = jnp.concatenate([src, pad_dummy]).reshape(rows2d)
    srcp_hop = jnp.concatenate([src, jnp.zeros((pad,), jnp.int32)]
                               ).reshape(rows2d)
    dstp = jnp.concatenate([dst, pad_dummy]).reshape(rows2d)
    Wp_pad = jnp.pad(Wp, ((0, 0), (0, DP - NCLS)))

    degp = _deg_call(srcp_deg, dstp)
    degp_t = degp.reshape(2 * NC, NPAD).T
    z0 = _mlp_call(h, W_emb, b_emb.reshape(1, HID), W1, b1.reshape(1, HID),
                   W2, b2.reshape(1, HID), Wp_pad)
    zs = _scale_call(degp_t, z0)
    p1 = _hop_call(zs, srcp_hop, dstp)
    zs2 = _mid_call(p1, degp_t)
    p2 = _hop_call(zs2, srcp_hop, dstp)
    return _fin_call(p2, degp_t, bp.reshape(1, NCLS))


# speedup vs baseline: 6.9268x; 1.0042x over previous
import functools

import jax
import jax.numpy as jnp
from jax import lax
from jax.experimental import pallas as pl
from jax.experimental.pallas import tpu as pltpu
from jax.experimental.pallas import tpu_sc as plsc

N = 10000
E = 160000
HID = 256
NCLS = 40
DP = 48
NPAD = 10240
NC = 2
NS = 16
NW = NC * NS
CHUNK = 128
EP = 163840
EPT = EP // NW
GCH = EPT // CHUNK
NROWS_T = NPAD // NS

_mesh = plsc.VectorSubcoreMesh(core_axis_name="c", subcore_axis_name="s")
_sc_params = pltpu.CompilerParams(use_tc_tiling_on_sc=False)


def _deg_body(srcp_hbm, dstp_hbm, out_hbm,
              sidx_v, didx_v, ones_v, zb_v, acc_o, acc_i, sem):
    cid = lax.axis_index("c")
    sid = lax.axis_index("s")
    wid = sid * NC + cid
    for k in range(CHUNK // 16):
        ones_v[pl.ds(k * 16, 16)] = jnp.full((16,), 1.0, jnp.float32)
    for k in range(NROWS_T // 16):
        zb_v[pl.ds(k * 16, 16)] = jnp.zeros((16,), jnp.float32)
    base_n = sid * NROWS_T
    pre = [pltpu.async_copy(srcp_hbm.at[pl.ds(wid * GCH, GCH), :], sidx_v, sem),
           pltpu.async_copy(dstp_hbm.at[pl.ds(wid * GCH, GCH), :], didx_v, sem)]
    pltpu.sync_copy(zb_v, acc_o.at[pl.ds(base_n, NROWS_T)])
    pltpu.sync_copy(zb_v, acc_i.at[pl.ds(base_n, NROWS_T)])
    for d in pre:
        d.wait()
    plsc.subcore_barrier()
    ds = []
    for g in range(GCH):
        ds.append(pltpu.async_copy(ones_v, acc_o.at[sidx_v.at[g]], sem,
                                   add=True))
        ds.append(pltpu.async_copy(ones_v, acc_i.at[didx_v.at[g]], sem,
                                   add=True))
    for d in ds:
        d.wait()
    plsc.subcore_barrier()
    pltpu.sync_copy(acc_o.at[pl.ds(base_n, NROWS_T)], zb_v)
    pltpu.sync_copy(zb_v, out_hbm.at[cid, 0, pl.ds(base_n, NROWS_T)])
    pltpu.sync_copy(acc_i.at[pl.ds(base_n, NROWS_T)], zb_v)
    pltpu.sync_copy(zb_v, out_hbm.at[cid, 1, pl.ds(base_n, NROWS_T)])


_deg_call = pl.kernel(
    _deg_body,
    out_type=jax.ShapeDtypeStruct((NC, 2, NPAD), jnp.float32),
    mesh=_mesh,
    scratch_types=[
        pltpu.VMEM((GCH, CHUNK), jnp.int32),
        pltpu.VMEM((GCH, CHUNK), jnp.int32),
        pltpu.VMEM((CHUNK,), jnp.float32),
        pltpu.VMEM((NROWS_T,), jnp.float32),
        pltpu.VMEM_SHARED((NPAD,), jnp.float32),
        pltpu.VMEM_SHARED((NPAD,), jnp.float32),
        pltpu.SemaphoreType.DMA,
    ],
    compiler_params=_sc_params,
)


NB = 8


def _hop_body(zs_hbm, srcp_hbm, dstp_hbm, out_hbm,
              sidx_v, didx_v, rows_v, acc, gsem, ssem):
    cid = lax.axis_index("c")
    sid = lax.axis_index("s")
    wid = sid * NC + cid

    pre = [pltpu.async_copy(srcp_hbm.at[pl.ds(wid * GCH, GCH), :], sidx_v,
                            gsem),
           pltpu.async_copy(dstp_hbm.at[pl.ds(wid * GCH, GCH), :], didx_v,
                            gsem)]

    def zrow(r, carry):
        for k in range(DP // 16):
            rows_v[0, r, pl.ds(k * 16, 16)] = jnp.zeros((16,), jnp.float32)
        return carry

    lax.fori_loop(0, CHUNK, zrow, 0)
    base_n = sid * NROWS_T
    for k in range(NROWS_T // CHUNK):
        pltpu.sync_copy(rows_v.at[0],
                        acc.at[pl.ds(base_n + k * CHUNK, CHUNK), :])
    for d in pre:
        d.wait()
    plsc.subcore_barrier()

    for grp in range(GCH // NB):
        gd = [pltpu.async_copy(zs_hbm.at[sidx_v.at[grp * NB + b]],
                               rows_v.at[b], gsem)
              for b in range(NB)]
        for d in gd:
            d.wait()
        sd = [pltpu.async_copy(rows_v.at[b],
                               acc.at[didx_v.at[grp * NB + b]], ssem,
                               add=True)
              for b in range(NB)]
        for d in sd:
            d.wait()

    plsc.subcore_barrier()
    wd = [pltpu.async_copy(acc.at[pl.ds(base_n + k * CHUNK, CHUNK), :],
                           rows_v.at[k], gsem)
          for k in range(NROWS_T // CHUNK)]
    for d in wd:
        d.wait()
    od = [pltpu.async_copy(rows_v.at[k],
                           out_hbm.at[cid, pl.ds(base_n + k * CHUNK, CHUNK),
                                      :], ssem)
          for k in range(NROWS_T // CHUNK)]
    for d in od:
        d.wait()


_hop_call = pl.kernel(
    _hop_body,
    out_type=jax.ShapeDtypeStruct((NC, NPAD, DP), jnp.float32),
    mesh=_mesh,
    scratch_types=[
        pltpu.VMEM((GCH, CHUNK), jnp.int32),
        pltpu.VMEM((GCH, CHUNK), jnp.int32),
        pltpu.VMEM((NB, CHUNK, DP), jnp.float32),
        pltpu.VMEM_SHARED((NPAD, DP), jnp.float32),
        pltpu.SemaphoreType.DMA,
        pltpu.SemaphoreType.DMA,
    ],
    compiler_params=_sc_params,
)


BLK = 1000


def _mlp_body(h_ref, we_ref, be_ref, w1_ref, b1_ref, w2_ref, b2_ref, wp_ref,
              out_ref):
    x = jnp.dot(h_ref[...], we_ref[...], preferred_element_type=jnp.float32)
    x = x + be_ref[...]
    x = jnp.dot(x, w1_ref[...], preferred_element_type=jnp.float32) + b1_ref[...]
    x = jnp.maximum(x, 0.0)
    x = jnp.dot(x, w2_ref[...], preferred_element_type=jnp.float32) + b2_ref[...]
    out_ref[...] = jnp.dot(x, wp_ref[...], preferred_element_type=jnp.float32)


def _norms(degp):
    no = lax.rsqrt(jnp.maximum(degp[:, 0] + degp[:, 2], 1.0))
    ni = lax.rsqrt(jnp.maximum(degp[:, 1] + degp[:, 3], 1.0))
    return no, ni


def _scale_body(degp_ref, z0_ref, out_ref):
    no, _ = _norms(degp_ref[...])
    out_ref[...] = z0_ref[...] * no[:, None]


def _mid_body(p_ref, degp_ref, out_ref):
    no, ni = _norms(degp_ref[...])
    p = p_ref[...]
    out_ref[...] = (p[0] + p[1]) * (ni * no)[:, None]


def _fin_body(p_ref, degp_ref, bp_ref, out_ref):
    _, ni = _norms(degp_ref[...])
    p = p_ref[...]
    y = (p[0] + p[1]) * ni[:, None]
    out_ref[...] = y[:, :NCLS] + bp_ref[...]


_full = lambda *shape: pl.BlockSpec(shape, lambda i: (0,) * len(shape))
_degp_spec = pl.BlockSpec((BLK, 4), lambda i: (i, 0))
_part_spec = pl.BlockSpec((NC, BLK, DP), lambda i: (0, i, 0))

_mlp_call = pl.pallas_call(
    _mlp_body,
    grid=(N // BLK,),
    in_specs=[
        pl.BlockSpec((BLK, HID), lambda i: (i, 0)),
        _full(HID, HID), _full(1, HID),
        _full(HID, HID), _full(1, HID),
        _full(HID, HID), _full(1, HID),
        _full(HID, DP),
    ],
    out_specs=pl.BlockSpec((BLK, DP), lambda i: (i, 0)),
    out_shape=jax.ShapeDtypeStruct((N, DP), jnp.float32),
)

_scale_call = pl.pallas_call(
    _scale_body,
    grid=(N // BLK,),
    in_specs=[_degp_spec, pl.BlockSpec((BLK, DP), lambda i: (i, 0))],
    out_specs=pl.BlockSpec((BLK, DP), lambda i: (i, 0)),
    out_shape=jax.ShapeDtypeStruct((N, DP), jnp.float32),
)

_mid_call = pl.pallas_call(
    _mid_body,
    grid=(N // BLK,),
    in_specs=[_part_spec, _degp_spec],
    out_specs=pl.BlockSpec((BLK, DP), lambda i: (i, 0)),
    out_shape=jax.ShapeDtypeStruct((N, DP), jnp.float32),
)

_fin_call = pl.pallas_call(
    _fin_body,
    grid=(N // BLK,),
    in_specs=[_part_spec, _degp_spec, _full(1, NCLS)],
    out_specs=pl.BlockSpec((BLK, NCLS), lambda i: (i, 0)),
    out_shape=jax.ShapeDtypeStruct((N, NCLS), jnp.float32),
)


def kernel(h, edge_index, e, snorm_n, snorm_e,
           W_emb, b_emb, W1, b1, W2, b2, Wp, bp):
    del e, snorm_n, snorm_e
    src = edge_index[0]
    dst = edge_index[1]
    pad = EP - E
    pad_dummy = N + (jnp.arange(pad, dtype=jnp.int32) % (NPAD - N))
    rows2d = (EP // CHUNK, CHUNK)
    srcp_deg = jnp.concatenate([src, pad_dummy]).reshape(rows2d)
    srcp_hop = jnp.concatenate([src, jnp.zeros((pad,), jnp.int32)]
                               ).reshape(rows2d)
    dstp = jnp.concatenate([dst, pad_dummy]).reshape(rows2d)
    Wp_pad = jnp.pad(Wp, ((0, 0), (0, DP - NCLS)))

    degp = _deg_call(srcp_deg, dstp)
    degp_t = degp.reshape(2 * NC, NPAD).T
    z0 = _mlp_call(h, W_emb, b_emb.reshape(1, HID), W1, b1.reshape(1, HID),
                   W2, b2.reshape(1, HID), Wp_pad)
    zs = _scale_call(degp_t, z0)
    p1 = _hop_call(zs, srcp_hop, dstp)
    zs2 = _mid_call(p1, degp_t)
    p2 = _hop_call(zs2, srcp_hop, dstp)
    return _fin_call(p2, degp_t, bp.reshape(1, NCLS))
